# Initial kernel scaffold; baseline (speedup 1.0000x reference)
import jax
import jax.numpy as jnp
from jax.experimental import pallas as pl


def kernel(source, target, message, x_e, weight):
    raise NotImplementedError("write your pallas kernel here")


# speedup vs baseline: 4.1354x; 4.1354x over previous
import functools

import jax
import jax.numpy as jnp
from jax import lax
from jax.experimental import pallas as pl
from jax.experimental.pallas import tpu as pltpu
from jax.experimental.pallas import tpu_sc as plsc


def _proj_body(x_ref, w_ref, a_ref, m_ref):
    i = pl.program_id(0)
    a = jnp.dot(x_ref[...], w_ref[...], preferred_element_type=jnp.float32)
    a_ref[...] = a
    bm = jnp.max(a, axis=0, keepdims=True)

    @pl.when(i == 0)
    def _():
        m_ref[...] = bm

    @pl.when(i > 0)
    def _():
        m_ref[...] = jnp.maximum(m_ref[...], bm)


def _proj(x, wbd, block_rows):
    rows, d = x.shape
    h = wbd.shape[1]
    grid = rows // block_rows
    return pl.pallas_call(
        _proj_body,
        grid=(grid,),
        in_specs=[
            pl.BlockSpec((block_rows, d), lambda i: (i, 0)),
            pl.BlockSpec((d, h), lambda i: (0, 0)),
        ],
        out_specs=[
            pl.BlockSpec((block_rows, h), lambda i: (i, 0)),
            pl.BlockSpec((1, h), lambda i: (0, 0)),
        ],
        out_shape=[
            jax.ShapeDtypeStruct((rows, h), jnp.float32),
            jax.ShapeDtypeStruct((1, h), jnp.float32),
        ],
    )(x, wbd)


def _exp_body(a_ref, bg_ref, m_ref, p_ref):
    s = a_ref[...] + bg_ref[...]
    s = jnp.where(s >= 0, s, 0.1 * s)
    p_ref[...] = jnp.exp(s - m_ref[...])


def _exp_leaky(a_flat, bg_flat, m_row, block_rows):
    rows, lanes = a_flat.shape
    grid = rows // block_rows
    return pl.pallas_call(
        _exp_body,
        grid=(grid,),
        in_specs=[
            pl.BlockSpec((block_rows, lanes), lambda i: (i, 0)),
            pl.BlockSpec((block_rows, lanes), lambda i: (i, 0)),
            pl.BlockSpec((1, lanes), lambda i: (0, 0)),
        ],
        out_specs=pl.BlockSpec((block_rows, lanes), lambda i: (i, 0)),
        out_shape=jax.ShapeDtypeStruct((rows, lanes), jnp.float32),
    )(a_flat, bg_flat, m_row)


def _final_body(msg_ref, p_ref, sg_ref, ex_ref, o_ref):
    r = p_ref[...] / sg_ref[...]
    o_ref[...] = msg_ref[...] * jnp.dot(
        r, ex_ref[...], preferred_element_type=jnp.float32
    )


def _final(msg, p, sg, expand, block_rows):
    rows, d = msg.shape
    h = p.shape[1]
    grid = rows // block_rows
    return pl.pallas_call(
        _final_body,
        grid=(grid,),
        in_specs=[
            pl.BlockSpec((block_rows, d), lambda i: (i, 0)),
            pl.BlockSpec((block_rows, h), lambda i: (i, 0)),
            pl.BlockSpec((block_rows, h), lambda i: (i, 0)),
            pl.BlockSpec((h, d), lambda i: (0, 0)),
        ],
        out_specs=pl.BlockSpec((block_rows, d), lambda i: (i, 0)),
        out_shape=jax.ShapeDtypeStruct((rows, d), jnp.float32),
    )(msg, p, sg, expand)


_CHUNK = 80


def _make_gather_rows(num_edges, h, num_workers):
    per_w = num_edges // num_workers
    n_chunks = per_w // _CHUNK
    mesh = plsc.VectorSubcoreMesh(core_axis_name="c", subcore_axis_name="s")

    @functools.partial(
        pl.kernel,
        mesh=mesh,
        out_type=jax.ShapeDtypeStruct((num_edges, h), jnp.float32),
        scratch_types=[
            pltpu.VMEM((_CHUNK,), jnp.int32),
            pltpu.VMEM((_CHUNK, h), jnp.float32),
            pltpu.SemaphoreType.DMA,
        ],
        compiler_params=pltpu.CompilerParams(use_tc_tiling_on_sc=False),
    )
    def gather_rows(idx_hbm, table_hbm, out_hbm, idx_v, rows_v, sem):
        wid = lax.axis_index("s") * 2 + lax.axis_index("c")
        base = wid * per_w

        def body(i, carry):
            off = base + i * _CHUNK
            pltpu.sync_copy(idx_hbm.at[pl.ds(off, _CHUNK)], idx_v)
            pltpu.async_copy(table_hbm.at[idx_v], rows_v, sem).wait()
            pltpu.sync_copy(rows_v, out_hbm.at[pl.ds(off, _CHUNK)])
            return carry

        lax.fori_loop(0, n_chunks, body, 0)

    return gather_rows


def _make_segment_sum(num_edges, h, n_pad, num_subcores):
    per_t = num_edges // num_subcores
    n_chunks = per_t // _CHUNK
    zrows = n_pad // num_subcores
    mesh = plsc.VectorSubcoreMesh(core_axis_name="c", subcore_axis_name="s")

    @functools.partial(
        pl.kernel,
        mesh=mesh,
        out_type=jax.ShapeDtypeStruct((num_edges, h), jnp.float32),
        scratch_types=[
            pltpu.VMEM((_CHUNK,), jnp.int32),
            pltpu.VMEM((_CHUNK, h), jnp.float32),
            pltpu.VMEM((zrows, h), jnp.float32),
            pltpu.VMEM_SHARED((n_pad, h), jnp.float32),
            pltpu.SemaphoreType.DMA,
        ],
        compiler_params=pltpu.CompilerParams(use_tc_tiling_on_sc=False),
    )
    def segment_sum(idx_hbm, p_hbm, zeros_hbm, sg_hbm, idx_v, row_v, z_v, s_sh, sem):
        cid = lax.axis_index("c")
        sid = lax.axis_index("s")

        @pl.when(cid == 0)
        def _():
            pltpu.sync_copy(zeros_hbm.at[pl.ds(sid * zrows, zrows)], z_v)
            pltpu.sync_copy(z_v, s_sh.at[pl.ds(sid * zrows, zrows)])
            plsc.subcore_barrier()

            base = sid * per_t

            def scat(i, carry):
                off = base + i * _CHUNK
                pltpu.sync_copy(idx_hbm.at[pl.ds(off, _CHUNK)], idx_v)
                pltpu.sync_copy(p_hbm.at[pl.ds(off, _CHUNK)], row_v)
                pltpu.sync_copy(row_v, s_sh.at[idx_v], add=True)
                return carry

            lax.fori_loop(0, n_chunks, scat, 0)
            plsc.subcore_barrier()

            def gath(i, carry):
                off = base + i * _CHUNK
                pltpu.sync_copy(idx_hbm.at[pl.ds(off, _CHUNK)], idx_v)
                pltpu.async_copy(s_sh.at[idx_v], row_v, sem).wait()
                pltpu.sync_copy(row_v, sg_hbm.at[pl.ds(off, _CHUNK)])
                return carry

            lax.fori_loop(0, n_chunks, gath, 0)

    return segment_sum


def kernel(source, target, message, x_e, weight):
    num_nodes, d = x_e.shape
    num_edges = message.shape[0]
    h = weight.shape[0]
    dh = d // h

    k = jnp.arange(d)
    mask = (k[:, None] // dh == jnp.arange(h)[None, :]).astype(jnp.float32)
    w1bd = mask * weight[:, :dh].reshape(d)[:, None]
    w2bd = mask * weight[:, dh:].reshape(d)[:, None]

    a, max_a = _proj(message, w1bd, block_rows=4000)
    b, max_b = _proj(x_e, w2bd, block_rows=num_nodes)

    t = max_a + max_b
    m = jnp.where(t >= 0, t, 0.1 * t)
    m_row = jnp.tile(m.reshape(h), d // h).reshape(1, d)

    info = plsc.get_sparse_core_info()
    num_workers = info.num_cores * info.num_subcores

    bg = _make_gather_rows(num_edges, h, num_workers)(target, b)

    flat_rows = num_edges * h // d
    p_flat = _exp_leaky(
        a.reshape(flat_rows, d), bg.reshape(flat_rows, d), m_row, block_rows=2000
    )
    p = p_flat.reshape(num_edges, h)

    n_pad = ((num_nodes + 1023) // 1024) * 1024
    zeros = jnp.zeros((n_pad, h), jnp.float32)
    sg = _make_segment_sum(num_edges, h, n_pad, info.num_subcores)(target, p, zeros)

    out = _final(message, p, sg, mask.T, block_rows=4000)
    return out


# speedup vs baseline: 6.1106x; 1.4776x over previous
import functools

import jax
import jax.numpy as jnp
from jax import lax
from jax.experimental import pallas as pl
from jax.experimental.pallas import tpu as pltpu
from jax.experimental.pallas import tpu_sc as plsc


def _proj_body(x_ref, w_ref, a_ref, m_ref):
    i = pl.program_id(0)
    a = jnp.dot(x_ref[...], w_ref[...], preferred_element_type=jnp.float32)
    a_ref[...] = a
    bm = jnp.max(a, axis=0, keepdims=True)

    @pl.when(i == 0)
    def _():
        m_ref[...] = bm

    @pl.when(i > 0)
    def _():
        m_ref[...] = jnp.maximum(m_ref[...], bm)


def _proj(x, wbd, block_rows):
    rows, d = x.shape
    h = wbd.shape[1]
    grid = rows // block_rows
    return pl.pallas_call(
        _proj_body,
        grid=(grid,),
        in_specs=[
            pl.BlockSpec((block_rows, d), lambda i: (i, 0)),
            pl.BlockSpec((d, h), lambda i: (0, 0)),
        ],
        out_specs=[
            pl.BlockSpec((block_rows, h), lambda i: (i, 0)),
            pl.BlockSpec((1, h), lambda i: (0, 0)),
        ],
        out_shape=[
            jax.ShapeDtypeStruct((rows, h), jnp.float32),
            jax.ShapeDtypeStruct((1, h), jnp.float32),
        ],
    )(x, wbd)


def _exp_body(a_ref, bg_ref, m_ref, p_ref):
    s = a_ref[...] + bg_ref[...]
    s = jnp.where(s >= 0, s, 0.1 * s)
    p_ref[...] = jnp.exp(s - m_ref[...])


def _exp_leaky(a_flat, bg_flat, m_row, block_rows):
    rows, lanes = a_flat.shape
    grid = rows // block_rows
    return pl.pallas_call(
        _exp_body,
        grid=(grid,),
        in_specs=[
            pl.BlockSpec((block_rows, lanes), lambda i: (i, 0)),
            pl.BlockSpec((block_rows, lanes), lambda i: (i, 0)),
            pl.BlockSpec((1, lanes), lambda i: (0, 0)),
        ],
        out_specs=pl.BlockSpec((block_rows, lanes), lambda i: (i, 0)),
        out_shape=jax.ShapeDtypeStruct((rows, lanes), jnp.float32),
    )(a_flat, bg_flat, m_row)


def _final_body(msg_ref, p_ref, sg_ref, ex_ref, o_ref):
    r = p_ref[...] / sg_ref[...]
    o_ref[...] = msg_ref[...] * jnp.dot(
        r, ex_ref[...], preferred_element_type=jnp.float32
    )


def _final(msg, p, sg, expand, block_rows):
    rows, d = msg.shape
    h = p.shape[1]
    grid = rows // block_rows
    return pl.pallas_call(
        _final_body,
        grid=(grid,),
        in_specs=[
            pl.BlockSpec((block_rows, d), lambda i: (i, 0)),
            pl.BlockSpec((block_rows, h), lambda i: (i, 0)),
            pl.BlockSpec((block_rows, h), lambda i: (i, 0)),
            pl.BlockSpec((h, d), lambda i: (0, 0)),
        ],
        out_specs=pl.BlockSpec((block_rows, d), lambda i: (i, 0)),
        out_shape=jax.ShapeDtypeStruct((rows, d), jnp.float32),
    )(msg, p, sg, expand)


_CHUNK = 80
_NBUF = 5


def _make_gather_rows(num_edges, h, num_workers):
    per_w = num_edges // num_workers
    n_chunks = per_w // _CHUNK
    mesh = plsc.VectorSubcoreMesh(core_axis_name="c", subcore_axis_name="s")

    @functools.partial(
        pl.kernel,
        mesh=mesh,
        out_type=jax.ShapeDtypeStruct((num_edges, h), jnp.float32),
        scratch_types=[pltpu.VMEM((per_w,), jnp.int32)]
        + [pltpu.VMEM((_CHUNK, h), jnp.float32) for _ in range(_NBUF)]
        + [pltpu.SemaphoreType.DMA for _ in range(2 * _NBUF)],
        compiler_params=pltpu.CompilerParams(use_tc_tiling_on_sc=False),
    )
    def gather_rows(idx_hbm, table_hbm, out_hbm, idx_all, *bufs):
        rows = bufs[:_NBUF]
        gsem = bufs[_NBUF : 2 * _NBUF]
        wsem = bufs[2 * _NBUF : 3 * _NBUF]
        wid = lax.axis_index("s") * 2 + lax.axis_index("c")
        base = wid * per_w

        pltpu.sync_copy(idx_hbm.at[pl.ds(base, per_w)], idx_all)
        for b in range(_NBUF):
            pltpu.async_copy(
                table_hbm.at[idx_all.at[pl.ds(b * _CHUNK, _CHUNK)]], rows[b], gsem[b]
            )

        @pl.loop(0, n_chunks, step=_NBUF)
        def _(g):
            for b in range(_NBUF):
                c = g + b
                pltpu.make_async_copy(
                    out_hbm.at[pl.ds(0, _CHUNK)], rows[b], gsem[b]
                ).wait()
                pltpu.async_copy(
                    rows[b], out_hbm.at[pl.ds(base + c * _CHUNK, _CHUNK)], wsem[b]
                )
            for b in range(_NBUF):
                c2 = g + b + _NBUF
                pltpu.make_async_copy(
                    rows[b], out_hbm.at[pl.ds(0, _CHUNK)], wsem[b]
                ).wait()

                @pl.when(c2 < n_chunks)
                def _():
                    pltpu.async_copy(
                        table_hbm.at[idx_all.at[pl.ds(c2 * _CHUNK, _CHUNK)]],
                        rows[b],
                        gsem[b],
                    )

    return gather_rows


def _make_segment_sum(num_edges, h, n_pad, num_subcores):
    per_t = num_edges // num_subcores
    n_chunks = per_t // _CHUNK
    zrows = n_pad // num_subcores
    mesh = plsc.VectorSubcoreMesh(core_axis_name="c", subcore_axis_name="s")

    @functools.partial(
        pl.kernel,
        mesh=mesh,
        out_type=jax.ShapeDtypeStruct((num_edges, h), jnp.float32),
        scratch_types=[
            pltpu.VMEM((per_t,), jnp.int32),
            pltpu.VMEM((zrows, h), jnp.float32),
            pltpu.VMEM_SHARED((n_pad, h), jnp.float32),
        ]
        + [pltpu.VMEM((_CHUNK,), jnp.int32) for _ in range(_NBUF)]
        + [pltpu.VMEM((_CHUNK, h), jnp.float32) for _ in range(_NBUF)]
        + [pltpu.SemaphoreType.DMA for _ in range(3 * _NBUF)],
        compiler_params=pltpu.CompilerParams(use_tc_tiling_on_sc=False),
    )
    def segment_sum(idx_hbm, p_hbm, zeros_hbm, sg_hbm, idx_all, z_v, s_sh, *bufs):
        idx_v = bufs[:_NBUF]
        row_v = bufs[_NBUF : 2 * _NBUF]
        isem = bufs[2 * _NBUF : 3 * _NBUF]
        psem = bufs[3 * _NBUF : 4 * _NBUF]
        ssem = bufs[4 * _NBUF : 5 * _NBUF]
        cid = lax.axis_index("c")
        sid = lax.axis_index("s")

        @pl.when(cid == 0)
        def _():
            pltpu.sync_copy(zeros_hbm.at[pl.ds(sid * zrows, zrows)], z_v)
            pltpu.sync_copy(z_v, s_sh.at[pl.ds(sid * zrows, zrows)])
            plsc.subcore_barrier()

            base = sid * per_t

            for b in range(_NBUF):
                off = base + b * _CHUNK
                pltpu.async_copy(idx_hbm.at[pl.ds(off, _CHUNK)], idx_v[b], isem[b])
                pltpu.async_copy(p_hbm.at[pl.ds(off, _CHUNK)], row_v[b], psem[b])

            @pl.loop(0, n_chunks, step=_NBUF)
            def _(g):
                for b in range(_NBUF):
                    pltpu.make_async_copy(
                        idx_hbm.at[pl.ds(0, _CHUNK)], idx_v[b], isem[b]
                    ).wait()
                    pltpu.make_async_copy(
                        p_hbm.at[pl.ds(0, _CHUNK)], row_v[b], psem[b]
                    ).wait()
                    pltpu.async_copy(row_v[b], s_sh.at[idx_v[b]], ssem[b], add=True)
                for b in range(_NBUF):
                    c2 = g + b + _NBUF
                    pltpu.make_async_copy(
                        row_v[b], s_sh.at[idx_v[b]], ssem[b]
                    ).wait()

                    @pl.when(c2 < n_chunks)
                    def _():
                        off = base + c2 * _CHUNK
                        pltpu.async_copy(
                            idx_hbm.at[pl.ds(off, _CHUNK)], idx_v[b], isem[b]
                        )
                        pltpu.async_copy(
                            p_hbm.at[pl.ds(off, _CHUNK)], row_v[b], psem[b]
                        )

            plsc.subcore_barrier()

            pltpu.sync_copy(idx_hbm.at[pl.ds(base, per_t)], idx_all)
            for b in range(_NBUF):
                pltpu.async_copy(
                    s_sh.at[idx_all.at[pl.ds(b * _CHUNK, _CHUNK)]], row_v[b], isem[b]
                )

            @pl.loop(0, n_chunks, step=_NBUF)
            def _(g):
                for b in range(_NBUF):
                    c = g + b
                    pltpu.make_async_copy(
                        sg_hbm.at[pl.ds(0, _CHUNK)], row_v[b], isem[b]
                    ).wait()
                    pltpu.async_copy(
                        row_v[b], sg_hbm.at[pl.ds(base + c * _CHUNK, _CHUNK)], psem[b]
                    )
                for b in range(_NBUF):
                    c2 = g + b + _NBUF
                    pltpu.make_async_copy(
                        row_v[b], sg_hbm.at[pl.ds(0, _CHUNK)], psem[b]
                    ).wait()

                    @pl.when(c2 < n_chunks)
                    def _():
                        pltpu.async_copy(
                            s_sh.at[idx_all.at[pl.ds(c2 * _CHUNK, _CHUNK)]],
                            row_v[b],
                            isem[b],
                        )

    return segment_sum


def kernel(source, target, message, x_e, weight):
    num_nodes, d = x_e.shape
    num_edges = message.shape[0]
    h = weight.shape[0]
    dh = d // h

    k = jnp.arange(d)
    mask = (k[:, None] // dh == jnp.arange(h)[None, :]).astype(jnp.float32)
    w1bd = mask * weight[:, :dh].reshape(d)[:, None]
    w2bd = mask * weight[:, dh:].reshape(d)[:, None]

    a, max_a = _proj(message, w1bd, block_rows=4000)
    b, max_b = _proj(x_e, w2bd, block_rows=num_nodes)

    t = max_a + max_b
    m = jnp.where(t >= 0, t, 0.1 * t)
    m_row = jnp.tile(m.reshape(h), d // h).reshape(1, d)

    info = plsc.get_sparse_core_info()
    num_workers = info.num_cores * info.num_subcores

    bg = _make_gather_rows(num_edges, h, num_workers)(target, b)

    flat_rows = num_edges * h // d
    p_flat = _exp_leaky(
        a.reshape(flat_rows, d), bg.reshape(flat_rows, d), m_row, block_rows=2000
    )
    p = p_flat.reshape(num_edges, h)

    n_pad = ((num_nodes + 1023) // 1024) * 1024
    zeros = jnp.zeros((n_pad, h), jnp.float32)
    sg = _make_segment_sum(num_edges, h, n_pad, info.num_subcores)(target, p, zeros)

    out = _final(message, p, sg, mask.T, block_rows=4000)
    return out
